# baseline (speedup 1.0000x reference)
import functools

import jax
import jax.numpy as jnp
from jax import lax
from jax.experimental import pallas as pl
from jax.experimental.pallas import tpu as pltpu
from jax.experimental.pallas import tpu_sc as plsc

_B, _H, _W = 16, 512, 512
_N = _B * _H * _W
_K = 128
_RB = 128

_NW = 32
_PPW = _N // _NW
_C = 8192
_NCHUNK = _PPW // _C


def _sc_gather_body(tbly_h, tblx_h, inst_h, gy_h, gx_h,
                    ty_v, tx_v, in_v, oy_v, ox_v):
    c = lax.axis_index("c")
    s = lax.axis_index("s")
    wid = s * 2 + c
    b = wid // 2
    pltpu.sync_copy(tbly_h.at[b], ty_v)
    pltpu.sync_copy(tblx_h.at[b], tx_v)
    base0 = wid * _PPW

    def chunk_body(ci, _):
        base = base0 + ci * _C
        pltpu.sync_copy(inst_h.at[pl.ds(base, _C)], in_v)

        def vec_body(i, _):
            idx = in_v[pl.ds(i * 16, 16)]
            icl = jnp.clip(idx, 0, 100)
            oy_v[pl.ds(i * 16, 16)] = plsc.load_gather(ty_v, [icl])
            ox_v[pl.ds(i * 16, 16)] = plsc.load_gather(tx_v, [icl])
            return 0

        lax.fori_loop(0, _C // 16, vec_body, 0, unroll=4)
        pltpu.sync_copy(oy_v, gy_h.at[pl.ds(base, _C)])
        pltpu.sync_copy(ox_v, gx_h.at[pl.ds(base, _C)])
        return 0

    lax.fori_loop(0, _NCHUNK, chunk_body, 0)


def _sc_gather(tbly, tblx, inst_flat):
    mesh = plsc.VectorSubcoreMesh(core_axis_name="c", subcore_axis_name="s")
    f = pl.kernel(
        _sc_gather_body,
        out_type=(
            jax.ShapeDtypeStruct((_N,), jnp.float32),
            jax.ShapeDtypeStruct((_N,), jnp.float32),
        ),
        mesh=mesh,
        compiler_params=pltpu.CompilerParams(needs_layout_passes=False),
        scratch_types=(
            pltpu.VMEM((_K,), jnp.float32),
            pltpu.VMEM((_K,), jnp.float32),
            pltpu.VMEM((_C,), jnp.int32),
            pltpu.VMEM((_C,), jnp.float32),
            pltpu.VMEM((_C,), jnp.float32),
        ),
    )
    return f(tbly, tblx, inst_flat)


def _tc_body(gy_ref, gx_ref, out_ref):
    j = pl.program_id(1)
    cy = gy_ref[0]
    cx = gx_ref[0]
    row = (j * _RB + lax.broadcasted_iota(jnp.int32, (_RB, _W), 0)
           ).astype(jnp.float32)
    col = lax.broadcasted_iota(jnp.int32, (_RB, _W), 1).astype(jnp.float32)
    x = cx - row
    y = cy - col
    mf = (cy > -9999.0).astype(jnp.float32)
    cmask = 1.0 - ((jnp.abs(x) < 3.0) & (jnp.abs(y) < 3.0)).astype(jnp.float32)
    r2 = x * x + y * y
    rc = jnp.sqrt(jnp.maximum(r2, 1e-12))
    theta = jnp.arctan2(y, x)
    inv = mf / rc
    out_ref[0, 0] = rc * mf
    out_ref[0, 1] = theta
    out_ref[0, 2] = y * inv
    out_ref[0, 3] = x * inv
    out_ref[0, 4] = cmask


@functools.partial(jax.jit, static_argnames=())
def kernel(instances, centers, batch_index):
    del batch_index
    inst_flat = instances.reshape(_N)
    neg = jnp.full((_B, 1), -10000.0, jnp.float32)
    pad = jnp.zeros((_B, _K - 101), jnp.float32)
    tbly = jnp.concatenate([neg, centers[:, :, 0], pad], axis=1)
    tblx = jnp.concatenate([neg, centers[:, :, 1], pad], axis=1)

    gy, gx = _sc_gather(tbly, tblx, inst_flat)
    gy = gy.reshape(_B, _H, _W)
    gx = gx.reshape(_B, _H, _W)

    out = pl.pallas_call(
        _tc_body,
        grid=(_B, _H // _RB),
        in_specs=[
            pl.BlockSpec((1, _RB, _W), lambda b, j: (b, j, 0)),
            pl.BlockSpec((1, _RB, _W), lambda b, j: (b, j, 0)),
        ],
        out_specs=pl.BlockSpec((1, 5, _RB, _W), lambda b, j: (b, 0, j, 0)),
        out_shape=jax.ShapeDtypeStruct((_B, 5, _H, _W), jnp.float32),
    )(gy, gx)
    return out


# speedup vs baseline: 1.8110x; 1.8110x over previous
import functools

import jax
import jax.numpy as jnp
from jax import lax
from jax.experimental import pallas as pl
from jax.experimental.pallas import tpu as pltpu
from jax.experimental.pallas import tpu_sc as plsc

_B, _H, _W = 16, 512, 512
_N = _B * _H * _W
_K = 128
_RB = 128

_NW = 32
_PPW = _N // _NW
_C = 16384
_NCHUNK = _PPW // _C


def _sc_gather_body(tbly_h, tblx_h, inst_h, gy_h, gx_h,
                    ty_v, tx_v,
                    in0, in1, oy0, oy1, ox0, ox1,
                    sem_i0, sem_i1, sem_y0, sem_y1, sem_x0, sem_x1):
    c = lax.axis_index("c")
    s = lax.axis_index("s")
    wid = s * 2 + c
    b = wid // 2
    pltpu.sync_copy(tbly_h.at[b], ty_v)
    pltpu.sync_copy(tblx_h.at[b], tx_v)
    base0 = wid * _PPW

    in_bufs, in_sems = (in0, in1), (sem_i0, sem_i1)
    oy_bufs, oy_sems = (oy0, oy1), (sem_y0, sem_y1)
    ox_bufs, ox_sems = (ox0, ox1), (sem_x0, sem_x1)

    h_in = [None] * _NCHUNK
    h_oy = [None] * _NCHUNK
    h_ox = [None] * _NCHUNK
    h_in[0] = pltpu.async_copy(inst_h.at[pl.ds(base0, _C)], in_bufs[0],
                               in_sems[0])
    for ci in range(_NCHUNK):
        cur = ci % 2
        base = base0 + ci * _C
        h_in[ci].wait()
        if ci + 1 < _NCHUNK:
            nxt = (ci + 1) % 2
            h_in[ci + 1] = pltpu.async_copy(
                inst_h.at[pl.ds(base + _C, _C)], in_bufs[nxt], in_sems[nxt])
        if ci >= 2:
            h_oy[ci - 2].wait()
            h_ox[ci - 2].wait()
        in_v, oy_v, ox_v = in_bufs[cur], oy_bufs[cur], ox_bufs[cur]

        @plsc.parallel_loop(0, _C // 16, unroll=8)
        def vec_body(i):
            idx = in_v[pl.ds(i * 16, 16)]
            oy_v[pl.ds(i * 16, 16)] = plsc.load_gather(ty_v, [idx])
            ox_v[pl.ds(i * 16, 16)] = plsc.load_gather(tx_v, [idx])

        h_oy[ci] = pltpu.async_copy(oy_v, gy_h.at[pl.ds(base, _C)],
                                    oy_sems[cur])
        h_ox[ci] = pltpu.async_copy(ox_v, gx_h.at[pl.ds(base, _C)],
                                    ox_sems[cur])
    h_oy[_NCHUNK - 2].wait()
    h_ox[_NCHUNK - 2].wait()
    h_oy[_NCHUNK - 1].wait()
    h_ox[_NCHUNK - 1].wait()


def _sc_gather(tbly, tblx, inst_flat):
    mesh = plsc.VectorSubcoreMesh(core_axis_name="c", subcore_axis_name="s")
    f = pl.kernel(
        _sc_gather_body,
        out_type=(
            jax.ShapeDtypeStruct((_N,), jnp.float32),
            jax.ShapeDtypeStruct((_N,), jnp.float32),
        ),
        mesh=mesh,
        compiler_params=pltpu.CompilerParams(needs_layout_passes=False),
        scratch_types=(
            pltpu.VMEM((_K,), jnp.float32),
            pltpu.VMEM((_K,), jnp.float32),
            pltpu.VMEM((_C,), jnp.int32),
            pltpu.VMEM((_C,), jnp.int32),
            pltpu.VMEM((_C,), jnp.float32),
            pltpu.VMEM((_C,), jnp.float32),
            pltpu.VMEM((_C,), jnp.float32),
            pltpu.VMEM((_C,), jnp.float32),
            pltpu.SemaphoreType.DMA,
            pltpu.SemaphoreType.DMA,
            pltpu.SemaphoreType.DMA,
            pltpu.SemaphoreType.DMA,
            pltpu.SemaphoreType.DMA,
            pltpu.SemaphoreType.DMA,
        ),
    )
    return f(tbly, tblx, inst_flat)


def _tc_body(gy_ref, gx_ref, out_ref):
    j = pl.program_id(1)
    cy = gy_ref[0]
    cx = gx_ref[0]
    row = (j * _RB + lax.broadcasted_iota(jnp.int32, (_RB, _W), 0)
           ).astype(jnp.float32)
    col = lax.broadcasted_iota(jnp.int32, (_RB, _W), 1).astype(jnp.float32)
    x = cx - row
    y = cy - col
    mf = (cy > -9999.0).astype(jnp.float32)
    cmask = 1.0 - ((jnp.abs(x) < 3.0) & (jnp.abs(y) < 3.0)).astype(jnp.float32)
    r2 = x * x + y * y
    rc = jnp.sqrt(jnp.maximum(r2, 1e-12))
    theta = jnp.arctan2(y, x)
    inv = mf / rc
    out_ref[0, 0] = rc * mf
    out_ref[0, 1] = theta
    out_ref[0, 2] = y * inv
    out_ref[0, 3] = x * inv
    out_ref[0, 4] = cmask


@functools.partial(jax.jit, static_argnames=())
def kernel(instances, centers, batch_index):
    del batch_index
    inst_flat = instances.reshape(_N)
    neg = jnp.full((_B, 1), -10000.0, jnp.float32)
    pad = jnp.zeros((_B, _K - 101), jnp.float32)
    tbly = jnp.concatenate([neg, centers[:, :, 0], pad], axis=1)
    tblx = jnp.concatenate([neg, centers[:, :, 1], pad], axis=1)

    gy, gx = _sc_gather(tbly, tblx, inst_flat)
    gy = gy.reshape(_B, _H, _W)
    gx = gx.reshape(_B, _H, _W)

    out = pl.pallas_call(
        _tc_body,
        grid=(_B, _H // _RB),
        in_specs=[
            pl.BlockSpec((1, _RB, _W), lambda b, j: (b, j, 0)),
            pl.BlockSpec((1, _RB, _W), lambda b, j: (b, j, 0)),
        ],
        out_specs=pl.BlockSpec((1, 5, _RB, _W), lambda b, j: (b, 0, j, 0)),
        out_shape=jax.ShapeDtypeStruct((_B, 5, _H, _W), jnp.float32),
    )(gy, gx)
    return out


# speedup vs baseline: 2.5791x; 1.4241x over previous
import functools

import jax
import jax.numpy as jnp
from jax import lax
from jax.experimental import pallas as pl
from jax.experimental.pallas import tpu as pltpu
from jax.experimental.pallas import tpu_sc as plsc

_B, _H, _W = 16, 512, 512
_K = 128
_RB = 128

_NW = 32
_HROWS = _B * _H // _NW
_CR = 32
_NCHUNK = _HROWS // _CR


def _sc_gather_body(tbly_h, tblx_h, inst_h, gy_h, gx_h,
                    ty_v, tx_v, in_b, oy_b, ox_b,
                    sem_i, sem_y, sem_x):
    c = lax.axis_index("c")
    s = lax.axis_index("s")
    wid = s * 2 + c
    b = wid // 2
    pltpu.sync_copy(tbly_h.at[pl.ds(b * _K, _K)], ty_v)
    pltpu.sync_copy(tblx_h.at[pl.ds(b * _K, _K)], tx_v)
    row0 = wid * _HROWS

    def in_copy(ci, par):
        return pltpu.make_async_copy(
            inst_h.at[pl.ds(row0 + ci * _CR, _CR)],
            in_b.at[pl.ds(par * _CR, _CR)], sem_i.at[par])

    def oy_copy(ci, par):
        return pltpu.make_async_copy(
            oy_b.at[pl.ds(par * _CR, _CR)],
            gy_h.at[pl.ds(row0 + ci * _CR, _CR)], sem_y.at[par])

    def ox_copy(ci, par):
        return pltpu.make_async_copy(
            ox_b.at[pl.ds(par * _CR, _CR)],
            gx_h.at[pl.ds(row0 + ci * _CR, _CR)], sem_x.at[par])

    in_copy(0, 0).start()

    def chunk_body(ci, carry):
        par = lax.rem(ci, 2)
        nxt = 1 - par
        in_copy(ci, par).wait()

        @pl.when(ci + 1 < _NCHUNK)
        def _():
            in_copy(ci + 1, nxt).start()

        @pl.when(ci >= 2)
        def _():
            oy_copy(ci - 2, par).wait()
            ox_copy(ci - 2, par).wait()

        @plsc.parallel_loop(0, _CR, unroll=2)
        def row_body(r):
            rr = par * _CR + r
            for g in range(_W // 16):
                cc = g * 16
                idx = in_b[rr, pl.ds(cc, 16)]
                oy_b[rr, pl.ds(cc, 16)] = plsc.load_gather(ty_v, [idx])
                ox_b[rr, pl.ds(cc, 16)] = plsc.load_gather(tx_v, [idx])

        oy_copy(ci, par).start()
        ox_copy(ci, par).start()
        return carry

    lax.fori_loop(0, _NCHUNK, chunk_body, 0)
    oy_copy(_NCHUNK - 2, 0).wait()
    ox_copy(_NCHUNK - 2, 0).wait()
    oy_copy(_NCHUNK - 1, 1).wait()
    ox_copy(_NCHUNK - 1, 1).wait()


def _sc_gather(tbly, tblx, inst):
    mesh = plsc.VectorSubcoreMesh(core_axis_name="c", subcore_axis_name="s")
    f = pl.kernel(
        _sc_gather_body,
        out_type=(
            jax.ShapeDtypeStruct((_B * _H, _W), jnp.float32),
            jax.ShapeDtypeStruct((_B * _H, _W), jnp.float32),
        ),
        mesh=mesh,
        compiler_params=pltpu.CompilerParams(needs_layout_passes=False),
        scratch_types=(
            pltpu.VMEM((_K,), jnp.float32),
            pltpu.VMEM((_K,), jnp.float32),
            pltpu.VMEM((2 * _CR, _W), jnp.int32),
            pltpu.VMEM((2 * _CR, _W), jnp.float32),
            pltpu.VMEM((2 * _CR, _W), jnp.float32),
            pltpu.SemaphoreType.DMA((2,)),
            pltpu.SemaphoreType.DMA((2,)),
            pltpu.SemaphoreType.DMA((2,)),
        ),
    )
    return f(tbly, tblx, inst)


def _tc_body(gy_ref, gx_ref, out_ref):
    j = pl.program_id(1)
    cy = gy_ref[0]
    cx = gx_ref[0]
    row = (j * _RB + lax.broadcasted_iota(jnp.int32, (_RB, _W), 0)
           ).astype(jnp.float32)
    col = lax.broadcasted_iota(jnp.int32, (_RB, _W), 1).astype(jnp.float32)
    x = cx - row
    y = cy - col
    mf = (cy > -9999.0).astype(jnp.float32)
    cmask = 1.0 - ((jnp.abs(x) < 3.0) & (jnp.abs(y) < 3.0)).astype(jnp.float32)
    r2 = x * x + y * y
    rc = jnp.sqrt(jnp.maximum(r2, 1e-12))
    theta = jnp.arctan2(y, x)
    inv = mf / rc
    out_ref[0, 0] = rc * mf
    out_ref[0, 1] = theta
    out_ref[0, 2] = y * inv
    out_ref[0, 3] = x * inv
    out_ref[0, 4] = cmask


@functools.partial(jax.jit, static_argnames=())
def kernel(instances, centers, batch_index):
    del batch_index
    inst = instances.reshape(_B * _H, _W)
    neg = jnp.full((_B, 1), -10000.0, jnp.float32)
    pad = jnp.zeros((_B, _K - 101), jnp.float32)
    tbly = jnp.concatenate([neg, centers[:, :, 0], pad], axis=1).reshape(-1)
    tblx = jnp.concatenate([neg, centers[:, :, 1], pad], axis=1).reshape(-1)

    gy, gx = _sc_gather(tbly, tblx, inst)
    gy = gy.reshape(_B, _H, _W)
    gx = gx.reshape(_B, _H, _W)

    out = pl.pallas_call(
        _tc_body,
        grid=(_B, _H // _RB),
        in_specs=[
            pl.BlockSpec((1, _RB, _W), lambda b, j: (b, j, 0)),
            pl.BlockSpec((1, _RB, _W), lambda b, j: (b, j, 0)),
        ],
        out_specs=pl.BlockSpec((1, 5, _RB, _W), lambda b, j: (b, 0, j, 0)),
        out_shape=jax.ShapeDtypeStruct((_B, 5, _H, _W), jnp.float32),
    )(gy, gx)
    return out


# speedup vs baseline: 2.7079x; 1.0499x over previous
import functools

import jax
import jax.numpy as jnp
from jax import lax
from jax.experimental import pallas as pl
from jax.experimental.pallas import tpu as pltpu
from jax.experimental.pallas import tpu_sc as plsc

_B, _H, _W = 16, 512, 512
_K = 128
_RB = 128

_NW = 32
_HROWS = _B * _H // _NW
_CR = 32
_NCHUNK = _HROWS // _CR


def _sc_gather_body(tbly_h, tblx_h, inst_h, gy_h, gx_h,
                    ty_v, tx_v, in_b, oy_b, ox_b,
                    sem_i, sem_y, sem_x):
    c = lax.axis_index("c")
    s = lax.axis_index("s")
    wid = s * 2 + c
    b = wid // 2
    pltpu.sync_copy(tbly_h.at[pl.ds(b * _K, _K)], ty_v)
    pltpu.sync_copy(tblx_h.at[pl.ds(b * _K, _K)], tx_v)
    row0 = wid * _HROWS

    def in_copy(ci, par):
        return pltpu.make_async_copy(
            inst_h.at[pl.ds(row0 + ci * _CR, _CR)],
            in_b.at[pl.ds(par * _CR, _CR)], sem_i.at[par])

    def oy_copy(ci, par):
        return pltpu.make_async_copy(
            oy_b.at[pl.ds(par * _CR, _CR)],
            gy_h.at[pl.ds(row0 + ci * _CR, _CR)], sem_y.at[par])

    def ox_copy(ci, par):
        return pltpu.make_async_copy(
            ox_b.at[pl.ds(par * _CR, _CR)],
            gx_h.at[pl.ds(row0 + ci * _CR, _CR)], sem_x.at[par])

    in_copy(0, 0).start()

    def chunk_body(ci, carry):
        par = lax.rem(ci, 2)
        nxt = 1 - par
        in_copy(ci, par).wait()

        @pl.when(ci + 1 < _NCHUNK)
        def _():
            in_copy(ci + 1, nxt).start()

        @pl.when(ci >= 2)
        def _():
            oy_copy(ci - 2, par).wait()
            ox_copy(ci - 2, par).wait()

        @plsc.parallel_loop(0, _CR, unroll=2)
        def row_body(r):
            rr = par * _CR + r
            for g in range(_W // 16):
                cc = g * 16
                idx = in_b[rr, pl.ds(cc, 16)]
                oy_b[rr, pl.ds(cc, 16)] = plsc.load_gather(ty_v, [idx])
                ox_b[rr, pl.ds(cc, 16)] = plsc.load_gather(tx_v, [idx])

        oy_copy(ci, par).start()
        ox_copy(ci, par).start()
        return carry

    lax.fori_loop(0, _NCHUNK, chunk_body, 0)
    oy_copy(_NCHUNK - 2, 0).wait()
    ox_copy(_NCHUNK - 2, 0).wait()
    oy_copy(_NCHUNK - 1, 1).wait()
    ox_copy(_NCHUNK - 1, 1).wait()


def _sc_gather(tbly, tblx, inst):
    mesh = plsc.VectorSubcoreMesh(core_axis_name="c", subcore_axis_name="s")
    f = pl.kernel(
        _sc_gather_body,
        out_type=(
            jax.ShapeDtypeStruct((_B * _H, _W), jnp.float32),
            jax.ShapeDtypeStruct((_B * _H, _W), jnp.float32),
        ),
        mesh=mesh,
        compiler_params=pltpu.CompilerParams(needs_layout_passes=False),
        scratch_types=(
            pltpu.VMEM((_K,), jnp.float32),
            pltpu.VMEM((_K,), jnp.float32),
            pltpu.VMEM((2 * _CR, _W), jnp.int32),
            pltpu.VMEM((2 * _CR, _W), jnp.float32),
            pltpu.VMEM((2 * _CR, _W), jnp.float32),
            pltpu.SemaphoreType.DMA((2,)),
            pltpu.SemaphoreType.DMA((2,)),
            pltpu.SemaphoreType.DMA((2,)),
        ),
    )
    return f(tbly, tblx, inst)


def _fast_atan2(y, x):
    ax = jnp.abs(x)
    ay = jnp.abs(y)
    mx = jnp.maximum(ax, ay)
    t = jnp.minimum(ax, ay) / jnp.maximum(mx, 1e-30)
    s = t * t
    p = t * (0.99921406 + s * (-0.32117747 + s * (0.14627053 + s * (-0.03899059))))
    p = jnp.where(ay > ax, 1.5707963267948966 - p, p)
    p = jnp.where(x < 0.0, 3.141592653589793 - p, p)
    return jnp.where(y < 0.0, -p, p)


def _tc_body(gy_ref, gx_ref, out_ref):
    j = pl.program_id(1)
    cy = gy_ref[0]
    cx = gx_ref[0]
    row = (j * _RB + lax.broadcasted_iota(jnp.int32, (_RB, _W), 0)
           ).astype(jnp.float32)
    col = lax.broadcasted_iota(jnp.int32, (_RB, _W), 1).astype(jnp.float32)
    x = cx - row
    y = cy - col
    mf = (cy > -9999.0).astype(jnp.float32)
    cmask = 1.0 - ((jnp.abs(x) < 3.0) & (jnp.abs(y) < 3.0)).astype(jnp.float32)
    r2 = x * x + y * y
    inv = lax.rsqrt(jnp.maximum(r2, 1e-12))
    minv = mf * inv
    out_ref[0, 0] = r2 * minv
    out_ref[0, 1] = _fast_atan2(y, x)
    out_ref[0, 2] = y * minv
    out_ref[0, 3] = x * minv
    out_ref[0, 4] = cmask


@functools.partial(jax.jit, static_argnames=())
def kernel(instances, centers, batch_index):
    del batch_index
    inst = instances.reshape(_B * _H, _W)
    neg = jnp.full((_B, 1), -10000.0, jnp.float32)
    pad = jnp.zeros((_B, _K - 101), jnp.float32)
    tbly = jnp.concatenate([neg, centers[:, :, 0], pad], axis=1).reshape(-1)
    tblx = jnp.concatenate([neg, centers[:, :, 1], pad], axis=1).reshape(-1)

    gy, gx = _sc_gather(tbly, tblx, inst)
    gy = gy.reshape(_B, _H, _W)
    gx = gx.reshape(_B, _H, _W)

    out = pl.pallas_call(
        _tc_body,
        grid=(_B, _H // _RB),
        in_specs=[
            pl.BlockSpec((1, _RB, _W), lambda b, j: (b, j, 0)),
            pl.BlockSpec((1, _RB, _W), lambda b, j: (b, j, 0)),
        ],
        out_specs=pl.BlockSpec((1, 5, _RB, _W), lambda b, j: (b, 0, j, 0)),
        out_shape=jax.ShapeDtypeStruct((_B, 5, _H, _W), jnp.float32),
    )(gy, gx)
    return out


# speedup vs baseline: 3.0310x; 1.1193x over previous
import functools

import jax
import jax.numpy as jnp
from jax import lax
from jax.experimental import pallas as pl
from jax.experimental.pallas import tpu as pltpu
from jax.experimental.pallas import tpu_sc as plsc

_B, _H, _W = 16, 512, 512
_K = 128
_RB = 128
_FP = 64.0

_NW = 32
_HROWS = _B * _H // _NW
_CR = 32
_NCHUNK = _HROWS // _CR


def _sc_gather_body(tbl_h, inst_h, gp_h, tb_v, in_b, out_b,
                    sem_i, sem_o):
    c = lax.axis_index("c")
    s = lax.axis_index("s")
    wid = s * 2 + c
    b = wid // 2
    pltpu.sync_copy(tbl_h.at[pl.ds(b * _K, _K)], tb_v)
    row0 = wid * _HROWS

    def in_copy(ci, par):
        return pltpu.make_async_copy(
            inst_h.at[pl.ds(row0 + ci * _CR, _CR)],
            in_b.at[pl.ds(par * _CR, _CR)], sem_i.at[par])

    def out_copy(ci, par):
        return pltpu.make_async_copy(
            out_b.at[pl.ds(par * _CR, _CR)],
            gp_h.at[pl.ds(row0 + ci * _CR, _CR)], sem_o.at[par])

    in_copy(0, 0).start()

    def chunk_body(ci, carry):
        par = lax.rem(ci, 2)
        nxt = 1 - par
        in_copy(ci, par).wait()

        @pl.when(ci + 1 < _NCHUNK)
        def _():
            in_copy(ci + 1, nxt).start()

        @pl.when(ci >= 2)
        def _():
            out_copy(ci - 2, par).wait()

        @plsc.parallel_loop(0, _CR, unroll=2)
        def row_body(r):
            rr = par * _CR + r
            for g in range(_W // 16):
                cc = g * 16
                idx = in_b[rr, pl.ds(cc, 16)]
                out_b[rr, pl.ds(cc, 16)] = plsc.load_gather(tb_v, [idx])

        out_copy(ci, par).start()
        return carry

    lax.fori_loop(0, _NCHUNK, chunk_body, 0)
    out_copy(_NCHUNK - 2, 0).wait()
    out_copy(_NCHUNK - 1, 1).wait()


def _sc_gather(tbl, inst):
    mesh = plsc.VectorSubcoreMesh(core_axis_name="c", subcore_axis_name="s")
    f = pl.kernel(
        _sc_gather_body,
        out_type=jax.ShapeDtypeStruct((_B * _H, _W), jnp.int32),
        mesh=mesh,
        compiler_params=pltpu.CompilerParams(needs_layout_passes=False),
        scratch_types=(
            pltpu.VMEM((_K,), jnp.int32),
            pltpu.VMEM((2 * _CR, _W), jnp.int32),
            pltpu.VMEM((2 * _CR, _W), jnp.int32),
            pltpu.SemaphoreType.DMA((2,)),
            pltpu.SemaphoreType.DMA((2,)),
        ),
    )
    return f(tbl, inst)


def _fast_atan2(y, x):
    ax = jnp.abs(x)
    ay = jnp.abs(y)
    mx = jnp.maximum(ax, ay)
    t = jnp.minimum(ax, ay) / jnp.maximum(mx, 1e-30)
    s = t * t
    p = t * (0.99921406 + s * (-0.32117747 + s * (0.14627053 + s * (-0.03899059))))
    p = jnp.where(ay > ax, 1.5707963267948966 - p, p)
    p = jnp.where(x < 0.0, 3.141592653589793 - p, p)
    return jnp.where(y < 0.0, -p, p)


def _tc_body(gp_ref, out_ref):
    j = pl.program_id(1)
    w = gp_ref[0]
    mask = w >= 0
    mf = mask.astype(jnp.float32)
    cy = jnp.where(mask, (w >> 16).astype(jnp.float32) * (1.0 / _FP),
                   -10000.0)
    cx = jnp.where(mask, (w & 0xFFFF).astype(jnp.float32) * (1.0 / _FP),
                   -10000.0)
    row = (j * _RB + lax.broadcasted_iota(jnp.int32, (_RB, _W), 0)
           ).astype(jnp.float32)
    col = lax.broadcasted_iota(jnp.int32, (_RB, _W), 1).astype(jnp.float32)
    x = cx - row
    y = cy - col
    cmask = 1.0 - ((jnp.abs(x) < 3.0) & (jnp.abs(y) < 3.0)).astype(jnp.float32)
    r2 = x * x + y * y
    inv = lax.rsqrt(jnp.maximum(r2, 1e-12))
    minv = mf * inv
    out_ref[0, 0] = r2 * minv
    out_ref[0, 1] = _fast_atan2(y, x)
    out_ref[0, 2] = y * minv
    out_ref[0, 3] = x * minv
    out_ref[0, 4] = cmask


@functools.partial(jax.jit, static_argnames=())
def kernel(instances, centers, batch_index):
    del batch_index
    inst = instances.reshape(_B * _H, _W)
    yq = jnp.round(centers[:, :, 0] * _FP).astype(jnp.int32)
    xq = jnp.round(centers[:, :, 1] * _FP).astype(jnp.int32)
    packed = (yq << 16) | xq
    neg = jnp.full((_B, 1), -1, jnp.int32)
    pad = jnp.full((_B, _K - 101), -1, jnp.int32)
    tbl = jnp.concatenate([neg, packed, pad], axis=1).reshape(-1)

    gp = _sc_gather(tbl, inst).reshape(_B, _H, _W)

    out = pl.pallas_call(
        _tc_body,
        grid=(_B, _H // _RB),
        in_specs=[
            pl.BlockSpec((1, _RB, _W), lambda b, j: (b, j, 0)),
        ],
        out_specs=pl.BlockSpec((1, 5, _RB, _W), lambda b, j: (b, 0, j, 0)),
        out_shape=jax.ShapeDtypeStruct((_B, 5, _H, _W), jnp.float32),
    )(gp)
    return out


# speedup vs baseline: 3.5676x; 1.1770x over previous
import functools

import jax
import jax.numpy as jnp
from jax import lax
from jax.experimental import pallas as pl
from jax.experimental.pallas import tpu as pltpu
from jax.experimental.pallas import tpu_sc as plsc

_B, _H, _W = 16, 512, 512
_K = 128
_RB = 256
_FP = 64.0

_NW = 32
_HROWS = _B * _H // _NW
_CR = 32
_NCHUNK = _HROWS // _CR


def _sc_gather_body(tbl_h, inst_h, gp_h, tb_v, in_b, out_b,
                    sem_i, sem_o):
    c = lax.axis_index("c")
    s = lax.axis_index("s")
    wid = s * 2 + c
    b = wid // 2
    pltpu.sync_copy(tbl_h.at[pl.ds(b * _K, _K)], tb_v)
    row0 = wid * _HROWS

    def in_copy(ci, par):
        return pltpu.make_async_copy(
            inst_h.at[pl.ds(row0 + ci * _CR, _CR)],
            in_b.at[pl.ds(par * _CR, _CR)], sem_i.at[par])

    def out_copy(ci, par):
        return pltpu.make_async_copy(
            out_b.at[pl.ds(par * _CR, _CR)],
            gp_h.at[pl.ds(row0 + ci * _CR, _CR)], sem_o.at[par])

    in_copy(0, 0).start()

    def chunk_body(ci, carry):
        par = lax.rem(ci, 2)
        nxt = 1 - par
        in_copy(ci, par).wait()

        @pl.when(ci + 1 < _NCHUNK)
        def _():
            in_copy(ci + 1, nxt).start()

        @pl.when(ci >= 2)
        def _():
            out_copy(ci - 2, par).wait()

        @plsc.parallel_loop(0, _CR, unroll=2)
        def row_body(r):
            rr = par * _CR + r
            for g in range(_W // 16):
                cc = g * 16
                idx = in_b[rr, pl.ds(cc, 16)]
                out_b[rr, pl.ds(cc, 16)] = plsc.load_gather(tb_v, [idx])

        out_copy(ci, par).start()
        return carry

    lax.fori_loop(0, _NCHUNK, chunk_body, 0)
    out_copy(_NCHUNK - 2, 0).wait()
    out_copy(_NCHUNK - 1, 1).wait()


def _sc_gather(tbl, inst):
    mesh = plsc.VectorSubcoreMesh(core_axis_name="c", subcore_axis_name="s")
    f = pl.kernel(
        _sc_gather_body,
        out_type=jax.ShapeDtypeStruct((_B * _H, _W), jnp.int32),
        mesh=mesh,
        compiler_params=pltpu.CompilerParams(needs_layout_passes=False),
        scratch_types=(
            pltpu.VMEM((_K,), jnp.int32),
            pltpu.VMEM((2 * _CR, _W), jnp.int32),
            pltpu.VMEM((2 * _CR, _W), jnp.int32),
            pltpu.SemaphoreType.DMA((2,)),
            pltpu.SemaphoreType.DMA((2,)),
        ),
    )
    return f(tbl, inst)


def _fast_atan2(y, x):
    ax = jnp.abs(x)
    ay = jnp.abs(y)
    mx = jnp.maximum(ax, ay)
    t = jnp.minimum(ax, ay) / jnp.maximum(mx, 1e-30)
    s = t * t
    p = t * (0.99921406 + s * (-0.32117747 + s * (0.14627053 + s * (-0.03899059))))
    p = jnp.where(ay > ax, 1.5707963267948966 - p, p)
    p = jnp.where(x < 0.0, 3.141592653589793 - p, p)
    return jnp.where(y < 0.0, -p, p)


def _tc_body(gp_ref, out_ref):
    j = pl.program_id(1)
    w = gp_ref[0]
    mask = w >= 0
    mf = mask.astype(jnp.float32)
    cy = jnp.where(mask, (w >> 16).astype(jnp.float32) * (1.0 / _FP),
                   -10000.0)
    cx = jnp.where(mask, (w & 0xFFFF).astype(jnp.float32) * (1.0 / _FP),
                   -10000.0)
    row = (j * _RB + lax.broadcasted_iota(jnp.int32, (_RB, _W), 0)
           ).astype(jnp.float32)
    col = lax.broadcasted_iota(jnp.int32, (_RB, _W), 1).astype(jnp.float32)
    x = cx - row
    y = cy - col
    cmask = 1.0 - ((jnp.abs(x) < 3.0) & (jnp.abs(y) < 3.0)).astype(jnp.float32)
    r2 = x * x + y * y
    inv = lax.rsqrt(jnp.maximum(r2, 1e-12))
    minv = mf * inv
    out_ref[0, 0] = r2 * minv
    out_ref[0, 1] = _fast_atan2(y, x)
    out_ref[0, 2] = y * minv
    out_ref[0, 3] = x * minv
    out_ref[0, 4] = cmask


@functools.partial(jax.jit, static_argnames=())
def kernel(instances, centers, batch_index):
    del batch_index
    inst = instances.reshape(_B * _H, _W)
    yq = jnp.round(centers[:, :, 0] * _FP).astype(jnp.int32)
    xq = jnp.round(centers[:, :, 1] * _FP).astype(jnp.int32)
    packed = (yq << 16) | xq
    neg = jnp.full((_B, 1), -1, jnp.int32)
    pad = jnp.full((_B, _K - 101), -1, jnp.int32)
    tbl = jnp.concatenate([neg, packed, pad], axis=1).reshape(-1)

    gp = _sc_gather(tbl, inst).reshape(_B, _H, _W)

    out = pl.pallas_call(
        _tc_body,
        grid=(_B, _H // _RB),
        in_specs=[
            pl.BlockSpec((1, _RB, _W), lambda b, j: (b, j, 0)),
        ],
        out_specs=pl.BlockSpec((1, 5, _RB, _W), lambda b, j: (b, 0, j, 0)),
        out_shape=jax.ShapeDtypeStruct((_B, 5, _H, _W), jnp.float32),
    )(gp)
    return out


# speedup vs baseline: 3.9337x; 1.1026x over previous
import functools

import jax
import jax.numpy as jnp
from jax import lax
from jax.experimental import pallas as pl
from jax.experimental.pallas import tpu as pltpu
from jax.experimental.pallas import tpu_sc as plsc

_B, _H, _W = 16, 512, 512
_K = 128
_RB = 512
_FP = 64.0

_NW = 32
_HROWS = _B * _H // _NW
_CR = 32
_NCHUNK = _HROWS // _CR


def _sc_gather_body(tbl_h, inst_h, gp_h, tb_v, in_b, out_b,
                    sem_i, sem_o):
    c = lax.axis_index("c")
    s = lax.axis_index("s")
    wid = s * 2 + c
    b = wid // 2
    pltpu.sync_copy(tbl_h.at[pl.ds(b * _K, _K)], tb_v)
    row0 = wid * _HROWS

    def in_copy(ci, par):
        return pltpu.make_async_copy(
            inst_h.at[pl.ds(row0 + ci * _CR, _CR)],
            in_b.at[pl.ds(par * _CR, _CR)], sem_i.at[par])

    def out_copy(ci, par):
        return pltpu.make_async_copy(
            out_b.at[pl.ds(par * _CR, _CR)],
            gp_h.at[pl.ds(row0 + ci * _CR, _CR)], sem_o.at[par])

    in_copy(0, 0).start()

    def chunk_body(ci, carry):
        par = lax.rem(ci, 2)
        nxt = 1 - par
        in_copy(ci, par).wait()

        @pl.when(ci + 1 < _NCHUNK)
        def _():
            in_copy(ci + 1, nxt).start()

        @pl.when(ci >= 2)
        def _():
            out_copy(ci - 2, par).wait()

        @plsc.parallel_loop(0, _CR, unroll=2)
        def row_body(r):
            rr = par * _CR + r
            for g in range(_W // 16):
                cc = g * 16
                idx = in_b[rr, pl.ds(cc, 16)]
                out_b[rr, pl.ds(cc, 16)] = plsc.load_gather(tb_v, [idx])

        out_copy(ci, par).start()
        return carry

    lax.fori_loop(0, _NCHUNK, chunk_body, 0)
    out_copy(_NCHUNK - 2, 0).wait()
    out_copy(_NCHUNK - 1, 1).wait()


def _sc_gather(tbl, inst):
    mesh = plsc.VectorSubcoreMesh(core_axis_name="c", subcore_axis_name="s")
    f = pl.kernel(
        _sc_gather_body,
        out_type=jax.ShapeDtypeStruct((_B * _H, _W), jnp.int32),
        mesh=mesh,
        compiler_params=pltpu.CompilerParams(needs_layout_passes=False),
        scratch_types=(
            pltpu.VMEM((_K,), jnp.int32),
            pltpu.VMEM((2 * _CR, _W), jnp.int32),
            pltpu.VMEM((2 * _CR, _W), jnp.int32),
            pltpu.SemaphoreType.DMA((2,)),
            pltpu.SemaphoreType.DMA((2,)),
        ),
    )
    return f(tbl, inst)


def _fast_atan2(y, x):
    ax = jnp.abs(x)
    ay = jnp.abs(y)
    mx = jnp.maximum(ax, ay)
    t = jnp.minimum(ax, ay) / jnp.maximum(mx, 1e-30)
    s = t * t
    p = t * (0.99921406 + s * (-0.32117747 + s * (0.14627053 + s * (-0.03899059))))
    p = jnp.where(ay > ax, 1.5707963267948966 - p, p)
    p = jnp.where(x < 0.0, 3.141592653589793 - p, p)
    return jnp.where(y < 0.0, -p, p)


def _tc_body(gp_ref, out_ref):
    j = pl.program_id(1)
    w = gp_ref[0]
    mask = w >= 0
    mf = mask.astype(jnp.float32)
    cy = jnp.where(mask, (w >> 16).astype(jnp.float32) * (1.0 / _FP),
                   -10000.0)
    cx = jnp.where(mask, (w & 0xFFFF).astype(jnp.float32) * (1.0 / _FP),
                   -10000.0)
    row = (j * _RB + lax.broadcasted_iota(jnp.int32, (_RB, _W), 0)
           ).astype(jnp.float32)
    col = lax.broadcasted_iota(jnp.int32, (_RB, _W), 1).astype(jnp.float32)
    x = cx - row
    y = cy - col
    cmask = 1.0 - ((jnp.abs(x) < 3.0) & (jnp.abs(y) < 3.0)).astype(jnp.float32)
    r2 = x * x + y * y
    inv = lax.rsqrt(jnp.maximum(r2, 1e-12))
    minv = mf * inv
    out_ref[0, 0] = r2 * minv
    out_ref[0, 1] = _fast_atan2(y, x)
    out_ref[0, 2] = y * minv
    out_ref[0, 3] = x * minv
    out_ref[0, 4] = cmask


@functools.partial(jax.jit, static_argnames=())
def kernel(instances, centers, batch_index):
    del batch_index
    inst = instances.reshape(_B * _H, _W)
    yq = jnp.round(centers[:, :, 0] * _FP).astype(jnp.int32)
    xq = jnp.round(centers[:, :, 1] * _FP).astype(jnp.int32)
    packed = (yq << 16) | xq
    neg = jnp.full((_B, 1), -1, jnp.int32)
    pad = jnp.full((_B, _K - 101), -1, jnp.int32)
    tbl = jnp.concatenate([neg, packed, pad], axis=1).reshape(-1)

    gp = _sc_gather(tbl, inst).reshape(_B, _H, _W)

    out = pl.pallas_call(
        _tc_body,
        grid=(_B, _H // _RB),
        in_specs=[
            pl.BlockSpec((1, _RB, _W), lambda b, j: (b, j, 0)),
        ],
        out_specs=pl.BlockSpec((1, 5, _RB, _W), lambda b, j: (b, 0, j, 0)),
        out_shape=jax.ShapeDtypeStruct((_B, 5, _H, _W), jnp.float32),
    )(gp)
    return out
